# Initial kernel scaffold; baseline (speedup 1.0000x reference)
import jax
import jax.numpy as jnp
from jax.experimental import pallas as pl


def kernel(x, edge_index, W1, a_src1, a_dst1, b1, W2, a_src2, a_dst2, b2):
    raise NotImplementedError("write your pallas kernel here")


# speedup vs baseline: 36.3700x; 36.3700x over previous
import functools

import jax
import jax.numpy as jnp
from jax import lax
from jax.experimental import pallas as pl
from jax.experimental.pallas import tpu as pltpu
from jax.experimental.pallas import tpu_sc as plsc

N = 10000
NP = 10240
H = 2
F = 128
D = 128
NTILES = 16
CHUNK = 128
NPT = NP // NTILES
NCHK = NPT // CHUNK


def _proj1_body(x_ref, w_ref, out_ref):
    out_ref[0] = jnp.dot(x_ref[...], w_ref[0],
                         preferred_element_type=jnp.float32)


def _tc_proj1(x_pad, w1r):
    nb = 10
    bn = NP // nb
    return pl.pallas_call(
        _proj1_body,
        grid=(H, nb),
        in_specs=[
            pl.BlockSpec((bn, D), lambda h, i: (i, 0)),
            pl.BlockSpec((1, D, F), lambda h, i: (h, 0, 0)),
        ],
        out_specs=pl.BlockSpec((1, bn, F), lambda h, i: (h, i, 0)),
        out_shape=jax.ShapeDtypeStruct((H, NP, F), jnp.float32),
    )(x_pad, w1r)


def _proj2_body(o_ref, w_ref, out_ref):
    acc = jnp.zeros((o_ref.shape[1], F), jnp.float32)
    for g in range(H):
        xg = o_ref[g]
        xg = jnp.where(xg > 0, xg, jnp.exp(jnp.minimum(xg, 0.0)) - 1.0)
        acc = acc + jnp.dot(xg, w_ref[g, 0],
                            preferred_element_type=jnp.float32)
    out_ref[0] = acc


def _tc_proj2(o1, w2r):
    nb = 10
    bn = NP // nb
    return pl.pallas_call(
        _proj2_body,
        grid=(H, nb),
        in_specs=[
            pl.BlockSpec((H, bn, F), lambda h, i: (0, i, 0)),
            pl.BlockSpec((H, 1, F, F), lambda h, i: (0, h, 0, 0)),
        ],
        out_specs=pl.BlockSpec((1, bn, F), lambda h, i: (h, i, 0)),
        out_shape=jax.ShapeDtypeStruct((H, NP, F), jnp.float32),
    )(o1, w2r)


def _make_sc_agg(cpt):
    mesh = plsc.VectorSubcoreMesh(core_axis_name="c", subcore_axis_name="s")

    @functools.partial(
        pl.kernel,
        out_type=jax.ShapeDtypeStruct((H * NP, F), jnp.float32),
        mesh=mesh,
        scratch_types=[
            pltpu.VMEM((NP,), jnp.float32),
            pltpu.VMEM((NP,), jnp.float32),
            pltpu.VMEM((NPT,), jnp.float32),
            pltpu.VMEM((NPT,), jnp.float32),
            pltpu.VMEM((CHUNK,), jnp.int32),
            pltpu.VMEM((CHUNK,), jnp.int32),
            pltpu.VMEM((CHUNK,), jnp.int32),
            pltpu.VMEM((CHUNK,), jnp.float32),
            pltpu.VMEM((CHUNK, F), jnp.float32),
            pltpu.VMEM((256,), jnp.float32),
            pltpu.VMEM((256,), jnp.float32),
            pltpu.VMEM((F,), jnp.float32),
            pltpu.VMEM((F,), jnp.float32),
            pltpu.VMEM((F,), jnp.float32),
            pltpu.VMEM_SHARED((NP, F), jnp.float32),
            pltpu.VMEM_SHARED((NP,), jnp.float32),
            pltpu.VMEM_SHARED((NP,), jnp.float32),
            pltpu.VMEM_SHARED((NP,), jnp.float32),
            pltpu.SemaphoreType.DMA,
        ],
        compiler_params=pltpu.CompilerParams(needs_layout_passes=False),
    )
    def sc_agg(hflat, aw_s, aw_d, bw, src, dst, o_hbm,
               asrc_tbl, adst_tbl, my_as, my_ad,
               sidx, didx, sidx2, wbuf, grows,
               stage_s, stage_d, avec_s, avec_d, bvec,
               num_sh, s_sh, as_sh, ad_sh, sem):
        hd = lax.axis_index("c")
        t = lax.axis_index("s")
        node_base = t * NPT
        hoff = hd * NP

        pltpu.sync_copy(aw_s.at[hd], avec_s)
        pltpu.sync_copy(aw_d.at[hd], avec_d)
        pltpu.sync_copy(bw.at[hd], bvec)

        def zero_row(r, _):
            for v in range(8):
                grows[r, pl.ds(v * 16, 16)] = jnp.zeros((16,), jnp.float32)
            return 0
        lax.fori_loop(0, CHUNK, zero_row, 0)
        for v in range(8):
            wbuf[pl.ds(v * 16, 16)] = jnp.zeros((16,), jnp.float32)
        for k in range(NCHK):
            pltpu.sync_copy(grows, num_sh.at[pl.ds(node_base + k * CHUNK, CHUNK)])
            pltpu.sync_copy(wbuf, s_sh.at[pl.ds(node_base + k * CHUNK, CHUNK)])

        iota16 = lax.iota(jnp.int32, 16)
        avs = [avec_s[pl.ds(v * 16, 16)] for v in range(8)]
        avd = [avec_d[pl.ds(v * 16, 16)] for v in range(8)]
        for k in range(NCHK):
            base = k * CHUNK
            pltpu.sync_copy(hflat.at[pl.ds(hoff + node_base + base, CHUNK)],
                            grows)

            def group_body(g, _):
                def row_body(j, _2):
                    r = g * 16 + j
                    acc_s = jnp.zeros((16,), jnp.float32)
                    acc_d = jnp.zeros((16,), jnp.float32)
                    for v in range(8):
                        hv = grows[r, pl.ds(v * 16, 16)]
                        acc_s = acc_s + hv * avs[v]
                        acc_d = acc_d + hv * avd[v]
                    stage_s[pl.ds(j * 16, 16)] = acc_s
                    stage_d[pl.ds(j * 16, 16)] = acc_d
                    return 0
                lax.fori_loop(0, 16, row_body, 0)
                tot_s = jnp.zeros((16,), jnp.float32)
                tot_d = jnp.zeros((16,), jnp.float32)
                for j in range(16):
                    idx = iota16 * 16 + j
                    tot_s = tot_s + plsc.load_gather(stage_s, [idx])
                    tot_d = tot_d + plsc.load_gather(stage_d, [idx])
                my_as[pl.ds(base + g * 16, 16)] = tot_s
                my_ad[pl.ds(base + g * 16, 16)] = tot_d
                return 0
            lax.fori_loop(0, 8, group_body, 0)

        pltpu.sync_copy(my_as, as_sh.at[pl.ds(node_base, NPT)])
        pltpu.sync_copy(my_ad, ad_sh.at[pl.ds(node_base, NPT)])
        plsc.subcore_barrier()
        pltpu.sync_copy(as_sh, asrc_tbl)
        pltpu.sync_copy(ad_sh, adst_tbl)

        def edge_chunk(c, _):
            off = (t * cpt + c) * CHUNK
            pltpu.sync_copy(src.at[pl.ds(off, CHUNK)], sidx)
            pltpu.sync_copy(dst.at[pl.ds(off, CHUNK)], didx)
            for g in range(8):
                sv = sidx[pl.ds(g * 16, 16)]
                dv = didx[pl.ds(g * 16, 16)]
                av = plsc.load_gather(asrc_tbl, [sv])
                bv = plsc.load_gather(adst_tbl, [dv])
                e = av + bv
                e = jnp.where(e >= 0, e, 0.2 * e)
                wbuf[pl.ds(g * 16, 16)] = jnp.exp(e)
                sidx2[pl.ds(g * 16, 16)] = sv + hoff
            pltpu.sync_copy(wbuf, s_sh.at[didx], add=True)
            pltpu.async_copy(hflat.at[sidx2], grows, sem).wait()

            def wgroup(g2, _):
                wv = wbuf[pl.ds(g2 * 16, 16)]
                for j in range(16):
                    r = g2 * 16 + j
                    w_r = wv[j]
                    for v in range(8):
                        grows[r, pl.ds(v * 16, 16)] = (
                            grows[r, pl.ds(v * 16, 16)] * w_r)
                return 0
            lax.fori_loop(0, 8, wgroup, 0)
            pltpu.sync_copy(grows, num_sh.at[didx], add=True)
            return 0
        lax.fori_loop(0, cpt, edge_chunk, 0)

        plsc.subcore_barrier()

        bvs = [bvec[pl.ds(v * 16, 16)] for v in range(8)]
        for k in range(NCHK):
            base = node_base + k * CHUNK
            pltpu.sync_copy(num_sh.at[pl.ds(base, CHUNK)], grows)
            pltpu.sync_copy(s_sh.at[pl.ds(base, CHUNK)], wbuf)

            def norm_group(g2, _):
                wv = wbuf[pl.ds(g2 * 16, 16)]
                rows = base + g2 * 16 + iota16
                mv = jnp.where(rows < N, 1.0, 0.0).astype(jnp.float32)
                srec_v = mv / (wv + 1e-16)
                for j in range(16):
                    r = g2 * 16 + j
                    m = mv[j]
                    srec = srec_v[j]
                    for v in range(8):
                        val = grows[r, pl.ds(v * 16, 16)] * srec + bvs[v] * m
                        grows[r, pl.ds(v * 16, 16)] = val
                return 0
            lax.fori_loop(0, 8, norm_group, 0)
            pltpu.sync_copy(grows, o_hbm.at[pl.ds(hoff + base, CHUNK)])

    return sc_agg


def kernel(x, edge_index, W1, a_src1, a_dst1, b1, W2, a_src2, a_dst2, b2):
    el = edge_index.shape[1] + N
    cpt = -(-el // (NTILES * CHUNK))
    ep = NTILES * cpt * CHUNK

    loops = jnp.arange(N, dtype=jnp.int32)
    src = jnp.concatenate([edge_index[0], loops,
                           jnp.full((ep - el,), NP - 1, jnp.int32)])
    dst = jnp.concatenate([edge_index[1], loops,
                           jnp.full((ep - el,), NP - 1, jnp.int32)])
    x_pad = jnp.pad(x, ((0, NP - N), (0, 0)))
    w1r = W1.reshape(D, H, F).transpose(1, 0, 2)
    w2r = W2.reshape(H, F, H, F).transpose(0, 2, 1, 3)
    b1r = b1.reshape(H, F)
    b2r = b2.reshape(H, F)

    sc_agg = _make_sc_agg(cpt)

    h1 = _tc_proj1(x_pad, w1r)
    o1 = sc_agg(h1.reshape(H * NP, F), a_src1, a_dst1, b1r, src, dst)
    h2 = _tc_proj2(o1.reshape(H, NP, F), w2r)
    o2 = sc_agg(h2.reshape(H * NP, F), a_src2, a_dst2, b2r, src, dst)
    o2 = o2.reshape(H, NP, F)
    return jnp.concatenate([o2[0, :N], o2[1, :N]], axis=1)


# speedup vs baseline: 40.5615x; 1.1152x over previous
import functools

import jax
import jax.numpy as jnp
from jax import lax
from jax.experimental import pallas as pl
from jax.experimental.pallas import tpu as pltpu
from jax.experimental.pallas import tpu_sc as plsc

N = 10000
NP = 10240
H = 2
F = 128
D = 128
NTILES = 16
CHUNK = 64
NPT = NP // NTILES
NCHK = NPT // CHUNK


def _proj1_body(x_ref, w_ref, out_ref):
    out_ref[0] = jnp.dot(x_ref[...], w_ref[0],
                         preferred_element_type=jnp.float32)


def _tc_proj1(x_pad, w1r):
    nb = 10
    bn = NP // nb
    return pl.pallas_call(
        _proj1_body,
        grid=(H, nb),
        in_specs=[
            pl.BlockSpec((bn, D), lambda h, i: (i, 0)),
            pl.BlockSpec((1, D, F), lambda h, i: (h, 0, 0)),
        ],
        out_specs=pl.BlockSpec((1, bn, F), lambda h, i: (h, i, 0)),
        out_shape=jax.ShapeDtypeStruct((H, NP, F), jnp.float32),
    )(x_pad, w1r)


def _proj2_body(o_ref, w_ref, out_ref):
    acc = jnp.zeros((o_ref.shape[1], F), jnp.float32)
    for g in range(H):
        xg = o_ref[g]
        xg = jnp.where(xg > 0, xg, jnp.exp(jnp.minimum(xg, 0.0)) - 1.0)
        acc = acc + jnp.dot(xg, w_ref[g, 0],
                            preferred_element_type=jnp.float32)
    out_ref[0] = acc


def _tc_proj2(o1, w2r):
    nb = 10
    bn = NP // nb
    return pl.pallas_call(
        _proj2_body,
        grid=(H, nb),
        in_specs=[
            pl.BlockSpec((H, bn, F), lambda h, i: (0, i, 0)),
            pl.BlockSpec((H, 1, F, F), lambda h, i: (0, h, 0, 0)),
        ],
        out_specs=pl.BlockSpec((1, bn, F), lambda h, i: (h, i, 0)),
        out_shape=jax.ShapeDtypeStruct((H, NP, F), jnp.float32),
    )(o1, w2r)


def _make_sc_agg(cpt):
    mesh = plsc.VectorSubcoreMesh(core_axis_name="c", subcore_axis_name="s")

    @functools.partial(
        pl.kernel,
        out_type=jax.ShapeDtypeStruct((H * NP, F), jnp.float32),
        mesh=mesh,
        scratch_types=[
            pltpu.VMEM((NP,), jnp.float32),
            pltpu.VMEM((NP,), jnp.float32),
            pltpu.VMEM((NPT,), jnp.float32),
            pltpu.VMEM((NPT,), jnp.float32),
            pltpu.VMEM((CHUNK,), jnp.int32),
            pltpu.VMEM((CHUNK,), jnp.int32),
            pltpu.VMEM((CHUNK,), jnp.int32),
            pltpu.VMEM((CHUNK,), jnp.int32),
            pltpu.VMEM((CHUNK,), jnp.int32),
            pltpu.VMEM((CHUNK,), jnp.int32),
            pltpu.VMEM((CHUNK,), jnp.float32),
            pltpu.VMEM((CHUNK,), jnp.float32),
            pltpu.VMEM((CHUNK, F), jnp.float32),
            pltpu.VMEM((CHUNK, F), jnp.float32),
            pltpu.VMEM((256,), jnp.float32),
            pltpu.VMEM((256,), jnp.float32),
            pltpu.VMEM((F,), jnp.float32),
            pltpu.VMEM((F,), jnp.float32),
            pltpu.VMEM((F,), jnp.float32),
            pltpu.VMEM_SHARED((NP, F), jnp.float32),
            pltpu.VMEM_SHARED((NP,), jnp.float32),
            pltpu.VMEM_SHARED((NP,), jnp.float32),
            pltpu.VMEM_SHARED((NP,), jnp.float32),
            pltpu.SemaphoreType.DMA,
            pltpu.SemaphoreType.DMA,
        ],
        compiler_params=pltpu.CompilerParams(needs_layout_passes=False),
    )
    def sc_agg(hflat, aw_s, aw_d, bw, src, dst, o_hbm,
               asrc_tbl, adst_tbl, my_as, my_ad,
               sidx0, sidx1, didx0, didx1, sidx2_0, sidx2_1,
               wbuf0, wbuf1, grows0, grows1,
               stage_s, stage_d, avec_s, avec_d, bvec,
               num_sh, s_sh, as_sh, ad_sh, sem0, sem1):
        sidx = (sidx0, sidx1)
        didx = (didx0, didx1)
        sidx2 = (sidx2_0, sidx2_1)
        wbuf = (wbuf0, wbuf1)
        grows = (grows0, grows1)
        sems = (sem0, sem1)
        hd = lax.axis_index("c")
        t = lax.axis_index("s")
        node_base = t * NPT
        hoff = hd * NP

        pltpu.sync_copy(aw_s.at[hd], avec_s)
        pltpu.sync_copy(aw_d.at[hd], avec_d)
        pltpu.sync_copy(bw.at[hd], bvec)

        def zero_row(r, _):
            for v in range(8):
                grows0[r, pl.ds(v * 16, 16)] = jnp.zeros((16,), jnp.float32)
            return 0
        lax.fori_loop(0, CHUNK, zero_row, 0)
        for v in range(CHUNK // 16):
            wbuf0[pl.ds(v * 16, 16)] = jnp.zeros((16,), jnp.float32)
        for k in range(NCHK):
            pltpu.sync_copy(grows0, num_sh.at[pl.ds(node_base + k * CHUNK, CHUNK)])
            pltpu.sync_copy(wbuf0, s_sh.at[pl.ds(node_base + k * CHUNK, CHUNK)])

        iota16 = lax.iota(jnp.int32, 16)
        avs = [avec_s[pl.ds(v * 16, 16)] for v in range(8)]
        avd = [avec_d[pl.ds(v * 16, 16)] for v in range(8)]
        for k in range(NCHK):
            base = k * CHUNK
            pltpu.sync_copy(hflat.at[pl.ds(hoff + node_base + base, CHUNK)],
                            grows0)

            def group_body(g, _):
                def row_body(j, _2):
                    r = g * 16 + j
                    acc_s = jnp.zeros((16,), jnp.float32)
                    acc_d = jnp.zeros((16,), jnp.float32)
                    for v in range(8):
                        hv = grows0[r, pl.ds(v * 16, 16)]
                        acc_s = acc_s + hv * avs[v]
                        acc_d = acc_d + hv * avd[v]
                    stage_s[pl.ds(j * 16, 16)] = acc_s
                    stage_d[pl.ds(j * 16, 16)] = acc_d
                    return 0
                lax.fori_loop(0, 16, row_body, 0)
                tot_s = jnp.zeros((16,), jnp.float32)
                tot_d = jnp.zeros((16,), jnp.float32)
                for j in range(16):
                    idx = iota16 * 16 + j
                    tot_s = tot_s + plsc.load_gather(stage_s, [idx])
                    tot_d = tot_d + plsc.load_gather(stage_d, [idx])
                my_as[pl.ds(base + g * 16, 16)] = tot_s
                my_ad[pl.ds(base + g * 16, 16)] = tot_d
                return 0
            lax.fori_loop(0, CHUNK // 16, group_body, 0)

        pltpu.sync_copy(my_as, as_sh.at[pl.ds(node_base, NPT)])
        pltpu.sync_copy(my_ad, ad_sh.at[pl.ds(node_base, NPT)])
        plsc.subcore_barrier()
        pltpu.sync_copy(as_sh, asrc_tbl)
        pltpu.sync_copy(ad_sh, adst_tbl)

        def prepare(b, c):
            @pl.when(c < cpt)
            def _():
                off = (t * cpt + c) * CHUNK
                pltpu.sync_copy(src.at[pl.ds(off, CHUNK)], sidx[b])
                pltpu.sync_copy(dst.at[pl.ds(off, CHUNK)], didx[b])
                for g in range(CHUNK // 16):
                    sv = sidx[b][pl.ds(g * 16, 16)]
                    dv = didx[b][pl.ds(g * 16, 16)]
                    av = plsc.load_gather(asrc_tbl, [sv])
                    bv = plsc.load_gather(adst_tbl, [dv])
                    e = av + bv
                    e = jnp.where(e >= 0, e, 0.2 * e)
                    wbuf[b][pl.ds(g * 16, 16)] = jnp.exp(e)
                    sidx2[b][pl.ds(g * 16, 16)] = sv + hoff
                pltpu.sync_copy(wbuf[b], s_sh.at[didx[b]], add=True)
                pltpu.async_copy(hflat.at[sidx2[b]], grows[b], sems[b])

        def finish(b):
            pltpu.make_async_copy(hflat.at[sidx2[b]], grows[b],
                                  sems[b]).wait()

            def wgroup(g2, _):
                wv = wbuf[b][pl.ds(g2 * 16, 16)]
                for j in range(16):
                    r = g2 * 16 + j
                    w_r = wv[j]
                    for v in range(8):
                        grows[b][r, pl.ds(v * 16, 16)] = (
                            grows[b][r, pl.ds(v * 16, 16)] * w_r)
                return 0
            lax.fori_loop(0, CHUNK // 16, wgroup, 0)
            pltpu.sync_copy(grows[b], num_sh.at[didx[b]], add=True)

        prepare(0, 0)

        def pipe_body(i, _):
            c0 = i * 2
            prepare(1, c0 + 1)
            finish(0)
            prepare(0, c0 + 2)
            finish(1)
            return 0
        lax.fori_loop(0, cpt // 2, pipe_body, 0)

        plsc.subcore_barrier()

        bvs = [bvec[pl.ds(v * 16, 16)] for v in range(8)]
        for k in range(NCHK):
            base = node_base + k * CHUNK
            pltpu.sync_copy(num_sh.at[pl.ds(base, CHUNK)], grows0)
            pltpu.sync_copy(s_sh.at[pl.ds(base, CHUNK)], wbuf0)

            def norm_group(g2, _):
                wv = wbuf0[pl.ds(g2 * 16, 16)]
                rows = base + g2 * 16 + iota16
                mv = jnp.where(rows < N, 1.0, 0.0).astype(jnp.float32)
                srec_v = mv / (wv + 1e-16)
                for j in range(16):
                    r = g2 * 16 + j
                    m = mv[j]
                    srec = srec_v[j]
                    for v in range(8):
                        val = grows0[r, pl.ds(v * 16, 16)] * srec + bvs[v] * m
                        grows0[r, pl.ds(v * 16, 16)] = val
                return 0
            lax.fori_loop(0, CHUNK // 16, norm_group, 0)
            pltpu.sync_copy(grows0, o_hbm.at[pl.ds(hoff + base, CHUNK)])

    return sc_agg


def kernel(x, edge_index, W1, a_src1, a_dst1, b1, W2, a_src2, a_dst2, b2):
    el = edge_index.shape[1] + N
    cpt = -(-el // (NTILES * CHUNK))
    cpt = cpt + (cpt % 2)
    ep = NTILES * cpt * CHUNK

    loops = jnp.arange(N, dtype=jnp.int32)
    src = jnp.concatenate([edge_index[0], loops,
                           jnp.full((ep - el,), NP - 1, jnp.int32)])
    dst = jnp.concatenate([edge_index[1], loops,
                           jnp.full((ep - el,), NP - 1, jnp.int32)])
    x_pad = jnp.pad(x, ((0, NP - N), (0, 0)))
    w1r = W1.reshape(D, H, F).transpose(1, 0, 2)
    w2r = W2.reshape(H, F, H, F).transpose(0, 2, 1, 3)
    b1r = b1.reshape(H, F)
    b2r = b2.reshape(H, F)

    sc_agg = _make_sc_agg(cpt)

    h1 = _tc_proj1(x_pad, w1r)
    o1 = sc_agg(h1.reshape(H * NP, F), a_src1, a_dst1, b1r, src, dst)
    h2 = _tc_proj2(o1.reshape(H, NP, F), w2r)
    o2 = sc_agg(h2.reshape(H * NP, F), a_src2, a_dst2, b2r, src, dst)
    o2 = o2.reshape(H, NP, F)
    return jnp.concatenate([o2[0, :N], o2[1, :N]], axis=1)


# speedup vs baseline: 41.9099x; 1.0332x over previous
import functools

import jax
import jax.numpy as jnp
from jax import lax
from jax.experimental import pallas as pl
from jax.experimental.pallas import tpu as pltpu
from jax.experimental.pallas import tpu_sc as plsc

N = 10000
NP = 10240
H = 2
F = 128
D = 128
NTILES = 16
CHUNK = 64
NPT = NP // NTILES
NCHK = NPT // CHUNK
BLK = 8


def _proj1_body(x_ref, w_ref, out_ref):
    out_ref[0] = jnp.dot(x_ref[...], w_ref[0],
                         preferred_element_type=jnp.float32)


def _tc_proj1(x_pad, w1r):
    nb = 10
    bn = NP // nb
    return pl.pallas_call(
        _proj1_body,
        grid=(H, nb),
        in_specs=[
            pl.BlockSpec((bn, D), lambda h, i: (i, 0)),
            pl.BlockSpec((1, D, F), lambda h, i: (h, 0, 0)),
        ],
        out_specs=pl.BlockSpec((1, bn, F), lambda h, i: (h, i, 0)),
        out_shape=jax.ShapeDtypeStruct((H, NP, F), jnp.float32),
    )(x_pad, w1r)


def _proj2_body(o_ref, w_ref, out_ref):
    acc = jnp.zeros((o_ref.shape[1], F), jnp.float32)
    for g in range(H):
        xg = o_ref[g]
        xg = jnp.where(xg > 0, xg, jnp.exp(jnp.minimum(xg, 0.0)) - 1.0)
        acc = acc + jnp.dot(xg, w_ref[g, 0],
                            preferred_element_type=jnp.float32)
    out_ref[0] = acc


def _tc_proj2(o1, w2r):
    nb = 10
    bn = NP // nb
    return pl.pallas_call(
        _proj2_body,
        grid=(H, nb),
        in_specs=[
            pl.BlockSpec((H, bn, F), lambda h, i: (0, i, 0)),
            pl.BlockSpec((H, 1, F, F), lambda h, i: (0, h, 0, 0)),
        ],
        out_specs=pl.BlockSpec((1, bn, F), lambda h, i: (h, i, 0)),
        out_shape=jax.ShapeDtypeStruct((H, NP, F), jnp.float32),
    )(o1, w2r)


def _make_sc_agg(cpt):
    mesh = plsc.VectorSubcoreMesh(core_axis_name="c", subcore_axis_name="s")

    @functools.partial(
        pl.kernel,
        out_type=jax.ShapeDtypeStruct((H * NP, F), jnp.float32),
        mesh=mesh,
        scratch_types=[
            pltpu.VMEM((NP,), jnp.float32),
            pltpu.VMEM((NP,), jnp.float32),
            pltpu.VMEM((NPT,), jnp.float32),
            pltpu.VMEM((NPT,), jnp.float32),
            pltpu.VMEM((BLK, CHUNK), jnp.int32),
            pltpu.VMEM((BLK, CHUNK), jnp.int32),
            pltpu.VMEM((CHUNK,), jnp.int32),
            pltpu.VMEM((CHUNK,), jnp.int32),
            pltpu.VMEM((CHUNK,), jnp.float32),
            pltpu.VMEM((CHUNK,), jnp.float32),
            pltpu.VMEM((CHUNK, F), jnp.float32),
            pltpu.VMEM((CHUNK, F), jnp.float32),
            pltpu.VMEM((256,), jnp.float32),
            pltpu.VMEM((256,), jnp.float32),
            pltpu.VMEM((F,), jnp.float32),
            pltpu.VMEM((F,), jnp.float32),
            pltpu.VMEM((F,), jnp.float32),
            pltpu.VMEM_SHARED((NP, F), jnp.float32),
            pltpu.VMEM_SHARED((NP,), jnp.float32),
            pltpu.VMEM_SHARED((NP,), jnp.float32),
            pltpu.VMEM_SHARED((NP,), jnp.float32),
            pltpu.SemaphoreType.DMA,
            pltpu.SemaphoreType.DMA,
        ],
        compiler_params=pltpu.CompilerParams(needs_layout_passes=False),
    )
    def sc_agg(hflat, aw_s, aw_d, bw, src2d, dst2d, o_hbm,
               asrc_tbl, adst_tbl, my_as, my_ad,
               sidx_blk, didx_blk, sidx2_0, sidx2_1,
               wbuf0, wbuf1, grows0, grows1,
               stage_s, stage_d, avec_s, avec_d, bvec,
               num_sh, s_sh, as_sh, ad_sh, sem0, sem1):
        sidx2 = (sidx2_0, sidx2_1)
        wbuf = (wbuf0, wbuf1)
        grows = (grows0, grows1)
        sems = (sem0, sem1)
        hd = lax.axis_index("c")
        t = lax.axis_index("s")
        node_base = t * NPT
        hoff = hd * NP

        pltpu.sync_copy(aw_s.at[hd], avec_s)
        pltpu.sync_copy(aw_d.at[hd], avec_d)
        pltpu.sync_copy(bw.at[hd], bvec)

        def zero_row(r, _):
            for v in range(8):
                grows0[r, pl.ds(v * 16, 16)] = jnp.zeros((16,), jnp.float32)
            return 0
        lax.fori_loop(0, CHUNK, zero_row, 0)
        for v in range(CHUNK // 16):
            wbuf0[pl.ds(v * 16, 16)] = jnp.zeros((16,), jnp.float32)
        for k in range(NCHK):
            pltpu.sync_copy(grows0, num_sh.at[pl.ds(node_base + k * CHUNK, CHUNK)])
            pltpu.sync_copy(wbuf0, s_sh.at[pl.ds(node_base + k * CHUNK, CHUNK)])

        iota16 = lax.iota(jnp.int32, 16)
        avs = [avec_s[pl.ds(v * 16, 16)] for v in range(8)]
        avd = [avec_d[pl.ds(v * 16, 16)] for v in range(8)]
        for k in range(NCHK):
            base = k * CHUNK
            pltpu.sync_copy(hflat.at[pl.ds(hoff + node_base + base, CHUNK)],
                            grows0)

            def group_body(g, _):
                def row_body(j, _2):
                    r = g * 16 + j
                    acc_s = jnp.zeros((16,), jnp.float32)
                    acc_d = jnp.zeros((16,), jnp.float32)
                    for v in range(8):
                        hv = grows0[r, pl.ds(v * 16, 16)]
                        acc_s = acc_s + hv * avs[v]
                        acc_d = acc_d + hv * avd[v]
                    stage_s[pl.ds(j * 16, 16)] = acc_s
                    stage_d[pl.ds(j * 16, 16)] = acc_d
                    return 0
                lax.fori_loop(0, 16, row_body, 0)
                tot_s = jnp.zeros((16,), jnp.float32)
                tot_d = jnp.zeros((16,), jnp.float32)
                for j in range(16):
                    idx = iota16 * 16 + j
                    tot_s = tot_s + plsc.load_gather(stage_s, [idx])
                    tot_d = tot_d + plsc.load_gather(stage_d, [idx])
                my_as[pl.ds(base + g * 16, 16)] = tot_s
                my_ad[pl.ds(base + g * 16, 16)] = tot_d
                return 0
            lax.fori_loop(0, CHUNK // 16, group_body, 0)

        pltpu.sync_copy(my_as, as_sh.at[pl.ds(node_base, NPT)])
        pltpu.sync_copy(my_ad, ad_sh.at[pl.ds(node_base, NPT)])
        plsc.subcore_barrier()
        pltpu.sync_copy(as_sh, asrc_tbl)
        pltpu.sync_copy(ad_sh, adst_tbl)

        nblk = cpt // BLK

        def prepare(b, jj):
            for g in range(CHUNK // 16):
                sv = sidx_blk[jj, pl.ds(g * 16, 16)]
                dv = didx_blk[jj, pl.ds(g * 16, 16)]
                av = plsc.load_gather(asrc_tbl, [sv])
                bv = plsc.load_gather(adst_tbl, [dv])
                e = av + bv
                e = jnp.where(e >= 0, e, 0.2 * e)
                wbuf[b][pl.ds(g * 16, 16)] = jnp.exp(e)
                sidx2[b][pl.ds(g * 16, 16)] = sv + hoff
            pltpu.sync_copy(wbuf[b], s_sh.at[didx_blk.at[jj]], add=True)
            pltpu.async_copy(hflat.at[sidx2[b]], grows[b], sems[b])

        def finish(b, jj):
            pltpu.make_async_copy(hflat.at[sidx2[b]], grows[b],
                                  sems[b]).wait()

            def wgroup(g2, _):
                wv = wbuf[b][pl.ds(g2 * 16, 16)]
                for j in range(16):
                    r = g2 * 16 + j
                    w_r = wv[j]
                    for v in range(8):
                        grows[b][r, pl.ds(v * 16, 16)] = (
                            grows[b][r, pl.ds(v * 16, 16)] * w_r)
                return 0
            lax.fori_loop(0, CHUNK // 16, wgroup, 0)
            pltpu.sync_copy(grows[b], num_sh.at[didx_blk.at[jj]], add=True)

        def block_body(blk, _):
            row0 = t * cpt + blk * BLK
            pltpu.sync_copy(src2d.at[pl.ds(row0, BLK)], sidx_blk)
            pltpu.sync_copy(dst2d.at[pl.ds(row0, BLK)], didx_blk)
            prepare(0, 0)
            for jj in range(BLK):
                if jj + 1 < BLK:
                    prepare((jj + 1) % 2, jj + 1)
                finish(jj % 2, jj)
            return 0
        lax.fori_loop(0, nblk, block_body, 0)

        plsc.subcore_barrier()

        bvs = [bvec[pl.ds(v * 16, 16)] for v in range(8)]
        for k in range(NCHK):
            base = node_base + k * CHUNK
            pltpu.sync_copy(num_sh.at[pl.ds(base, CHUNK)], grows0)
            pltpu.sync_copy(s_sh.at[pl.ds(base, CHUNK)], wbuf0)

            def norm_group(g2, _):
                wv = wbuf0[pl.ds(g2 * 16, 16)]
                rows = base + g2 * 16 + iota16
                mv = jnp.where(rows < N, 1.0, 0.0).astype(jnp.float32)
                srec_v = mv / (wv + 1e-16)
                for j in range(16):
                    r = g2 * 16 + j
                    m = mv[j]
                    srec = srec_v[j]
                    for v in range(8):
                        val = grows0[r, pl.ds(v * 16, 16)] * srec + bvs[v] * m
                        grows0[r, pl.ds(v * 16, 16)] = val
                return 0
            lax.fori_loop(0, CHUNK // 16, norm_group, 0)
            pltpu.sync_copy(grows0, o_hbm.at[pl.ds(hoff + base, CHUNK)])

    return sc_agg


def kernel(x, edge_index, W1, a_src1, a_dst1, b1, W2, a_src2, a_dst2, b2):
    el = edge_index.shape[1] + N
    cpt = -(-el // (NTILES * CHUNK))
    cpt = -(-cpt // BLK) * BLK
    ep = NTILES * cpt * CHUNK

    loops = jnp.arange(N, dtype=jnp.int32)
    src = jnp.concatenate([edge_index[0], loops,
                           jnp.full((ep - el,), NP - 1, jnp.int32)])
    dst = jnp.concatenate([edge_index[1], loops,
                           jnp.full((ep - el,), NP - 1, jnp.int32)])
    x_pad = jnp.pad(x, ((0, NP - N), (0, 0)))
    w1r = W1.reshape(D, H, F).transpose(1, 0, 2)
    w2r = W2.reshape(H, F, H, F).transpose(0, 2, 1, 3)
    b1r = b1.reshape(H, F)
    b2r = b2.reshape(H, F)

    sc_agg = _make_sc_agg(cpt)

    src2d = src.reshape(-1, CHUNK)
    dst2d = dst.reshape(-1, CHUNK)

    h1 = _tc_proj1(x_pad, w1r)
    o1 = sc_agg(h1.reshape(H * NP, F), a_src1, a_dst1, b1r, src2d, dst2d)
    h2 = _tc_proj2(o1.reshape(H, NP, F), w2r)
    o2 = sc_agg(h2.reshape(H * NP, F), a_src2, a_dst2, b2r, src2d, dst2d)
    o2 = o2.reshape(H, NP, F)
    return jnp.concatenate([o2[0, :N], o2[1, :N]], axis=1)
